# Initial kernel scaffold; baseline (speedup 1.0000x reference)
import jax
import jax.numpy as jnp
from jax.experimental import pallas as pl


def kernel(hs_last, w_sub1, b_sub1, w_sub2, b_sub2, w_obj1, b_obj1, w_obj2, b_obj2, conv1_w, conv1_b, conv2_w, conv2_b, w_pair1, b_pair1, w_pair2, b_pair2):
    raise NotImplementedError("write your pallas kernel here")


# speedup vs baseline: 9.7060x; 9.7060x over previous
import functools

import jax
import jax.numpy as jnp
from jax import lax
from jax.experimental import pallas as pl
from jax.experimental.pallas import tpu as pltpu
from jax.experimental.pallas import tpu_sc as plsc

B, Q, D, KPAIR = 16, 1024, 256, 256
N = Q // 2
K = KPAIR
GBUF = 256
TBUF = 512
NSUB = 32
WR = 32
MININT = -2147483648


def _skey(x):
    b = lax.bitcast_convert_type(x, jnp.int32)
    return jnp.where(b >= 0, b, b ^ jnp.int32(0x7FFFFFFF))


def _mlp_body(subf_ref, objf_ref, ws1, bs1, ws2, bs2, wo1, bo1, wo2, bo2,
              out_ref):
    dn = (((1,), (1,)), ((), ()))
    sub = subf_ref[...]
    obj = objf_ref[...]
    se = lax.dot_general(jnp.maximum(lax.dot_general(sub, ws1[...], dn)
                                     + bs1[...], 0.0), ws2[...], dn) + bs2[...]
    oe = lax.dot_general(jnp.maximum(lax.dot_general(obj, wo1[...], dn)
                                     + bo1[...], 0.0), wo2[...], dn) + bo2[...]
    se = se / (jnp.sqrt(jnp.sum(se * se, axis=1, keepdims=True)) + 1e-6)
    oe = oe / (jnp.sqrt(jnp.sum(oe * oe, axis=1, keepdims=True)) + 1e-6)
    out_ref[...] = lax.dot_general(se, oe, dn)


def _mlp_call(subf, objf, ws1, bs1, ws2, bs2, wo1, bo1, wo2, bo2):
    full = lambda s: pl.BlockSpec(s, lambda b: (0,) * len(s))
    return pl.pallas_call(
        _mlp_body,
        grid=(B,),
        in_specs=[
            pl.BlockSpec((None, N, D), lambda b: (b, 0, 0)),
            pl.BlockSpec((None, N, D), lambda b: (b, 0, 0)),
            full((D, D)), full((1, D)), full((D, D)), full((1, D)),
            full((D, D)), full((1, D)), full((D, D)), full((1, D)),
        ],
        out_specs=pl.BlockSpec((None, N, N), lambda b: (b, 0, 0)),
        out_shape=jax.ShapeDtypeStruct((B, N, N), jnp.float32),
    )(subf, objf, ws1, bs1, ws2, bs2, wo1, bo1, wo2, bo2)


def _conv_body(imp_ref, cw1, cb1, cw2, cb2, out_ref, xp_ref, acc_ref):
    imp = imp_ref[...].astype(jnp.bfloat16).astype(jnp.float32)
    zr = jnp.zeros((1, N), jnp.float32)
    xv = jnp.concatenate([zr, imp, zr], axis=0)
    zc = jnp.zeros((N + 2, 1), jnp.float32)
    xp_ref[...] = jnp.concatenate([zc, xv, zc], axis=1)
    acc_ref[...] = jnp.zeros((N, N), jnp.float32)

    def ch_body(c, _):
        y = jnp.zeros((N, N), jnp.float32)
        for a in range(3):
            for bb in range(3):
                w = cw1[c, a * 3 + bb].astype(jnp.bfloat16).astype(jnp.float32)
                y = y + w * xp_ref[a:a + N, bb:bb + N]
        r = jnp.maximum(y + cb1[c], 0.0)
        rb = r.astype(jnp.bfloat16).astype(jnp.float32)
        v = cw2[c].astype(jnp.bfloat16).astype(jnp.float32)
        acc_ref[...] = acc_ref[...] + v * rb
        return 0

    lax.fori_loop(0, 16, ch_body, 0)
    out_ref[...] = _skey(acc_ref[...] + cb2[0])


def _conv_call(imp, cw1, cb1, cw2, cb2):
    smem = lambda s: pl.BlockSpec(memory_space=pltpu.SMEM)
    return pl.pallas_call(
        _conv_body,
        grid=(B,),
        in_specs=[
            pl.BlockSpec((None, N, N), lambda b: (b, 0, 0)),
            smem((16, 9)), smem((16,)), smem((16,)), smem((1,)),
        ],
        out_specs=pl.BlockSpec((None, N, N), lambda b: (b, 0, 0)),
        out_shape=jax.ShapeDtypeStruct((B, N, N), jnp.int32),
        scratch_shapes=[pltpu.VMEM((N + 2, N + 2), jnp.float32),
                        pltpu.VMEM((N, N), jnp.float32)],
    )(imp, cw1, cb1, cw2, cb2)


def _thresh_body(sk_ref, t_ref):
    def bit_body(i, tu):
        bit = lax.shift_left(jnp.int32(1), 31 - i)
        cand_u = tu | bit
        cand_s = cand_u ^ MININT
        cnt = jnp.sum((sk_ref[...] >= cand_s[:, None, None])
                      .astype(jnp.int32), axis=(1, 2))
        return jnp.where(cnt >= K, cand_u, tu)

    tu = lax.fori_loop(0, 32, bit_body, jnp.zeros((B,), jnp.int32))
    ts = tu ^ MININT
    t_ref[...] = jnp.broadcast_to(ts[:, None], (B, 16))


def _thresh_call(skeys):
    return pl.pallas_call(
        _thresh_body,
        out_shape=jax.ShapeDtypeStruct((B, 16), jnp.int32),
    )(skeys)


def _sc_body(sk_hbm, tv_hbm, gti_hbm, gtk_hbm, tie_hbm,
             win_v, t_v, gti_v, gtk_v, tie_v, cnt_v):
    wid = lax.axis_index("s") * 2 + lax.axis_index("c")
    pltpu.sync_copy(tv_hbm.at[wid], t_v)
    t = t_v[...]
    iota = lax.broadcasted_iota(jnp.int32, (16,), 0)

    for i in range(GBUF // 16):
        gtk_v[pl.ds(i * 16, 16)] = jnp.full((16,), MININT, jnp.int32)
        gti_v[pl.ds(i * 16, 16)] = jnp.zeros((16,), jnp.int32)
    for i in range(TBUF // 16):
        tie_v[pl.ds(i * 16, 16)] = jnp.full((16,), 1 << 20, jnp.int32)

    cnt_v[pl.ds(0, 16)] = jnp.zeros((16,), jnp.int32)
    cnt_v[pl.ds(16, 16)] = jnp.zeros((16,), jnp.int32)
    half_base = (wid % 2) * (N // 2) * N

    one = jnp.full((16,), 1, jnp.int32)
    zero = jnp.zeros((16,), jnp.int32)

    def win_body(wi, _):
        pltpu.sync_copy(sk_hbm.at[wid, pl.ds(wi * WR, WR)], win_v)

        def row_body(r, _):
            row_base = half_base + (wi * WR + r) * N

            def vec_body(j, _):
                k = win_v[r, pl.ds(j * 16, 16)]
                idxv = jnp.full((16,), row_base + j * 16, jnp.int32) + iota
                gc = cnt_v[pl.ds(0, 16)]
                tc = cnt_v[pl.ds(16, 16)]
                m_gt = k > t
                cs = plsc.cumsum(jnp.where(m_gt, one, zero))
                pc = plsc.all_reduce_population_count(m_gt)
                pos = gc + cs - 1
                plsc.store_scatter(gti_v, [pos], idxv, mask=m_gt)
                plsc.store_scatter(gtk_v, [pos], k, mask=m_gt)
                m_eq = (k == t) & (tc < TBUF - 16)
                cse = plsc.cumsum(jnp.where(m_eq, one, zero))
                pe = plsc.all_reduce_population_count(m_eq)
                plsc.store_scatter(tie_v, [tc + cse - 1], idxv, mask=m_eq)
                cnt_v[pl.ds(0, 16)] = gc + pc
                cnt_v[pl.ds(16, 16)] = tc + pe
                return 0

            return lax.fori_loop(0, N // 16, vec_body, 0)

        return lax.fori_loop(0, WR, row_body, 0)

    lax.fori_loop(0, (N // 2) // WR, win_body, 0)

    pltpu.sync_copy(gti_v, gti_hbm.at[wid])
    pltpu.sync_copy(gtk_v, gtk_hbm.at[wid])
    pltpu.sync_copy(tie_v, tie_hbm.at[wid])


def _sc_call(sk32, tvec):
    mesh = plsc.VectorSubcoreMesh(core_axis_name="c", subcore_axis_name="s")
    f = functools.partial(
        pl.kernel,
        mesh=mesh,
        compiler_params=pltpu.CompilerParams(
            needs_layout_passes=False, use_tc_tiling_on_sc=False),
        out_type=[
            jax.ShapeDtypeStruct((NSUB, GBUF), jnp.int32),
            jax.ShapeDtypeStruct((NSUB, GBUF), jnp.int32),
            jax.ShapeDtypeStruct((NSUB, TBUF), jnp.int32),
        ],
        scratch_types=[
            pltpu.VMEM((WR, N), jnp.int32),
            pltpu.VMEM((16,), jnp.int32),
            pltpu.VMEM((GBUF,), jnp.int32),
            pltpu.VMEM((GBUF,), jnp.int32),
            pltpu.VMEM((TBUF,), jnp.int32),
            pltpu.VMEM((32,), jnp.int32),
        ],
    )(_sc_body)
    return f(sk32, tvec)


P = 2 * (GBUF + TBUF)
JC = 384


def _select_body(krow_ref, kcol_ref, irow_ref, icol_ref, subf_ref, objf_ref,
                 w1a, w1b, b1, w2, b2, out_ref):
    kr = krow_ref[...]
    ir = irow_ref[...]
    rank = jnp.zeros((1, P), jnp.int32)
    for jc in range(P // JC):
        kc = lax.slice(kcol_ref[...], (jc * JC, 0), ((jc + 1) * JC, 1))
        ic = lax.slice(icol_ref[...], (jc * JC, 0), ((jc + 1) * JC, 1))
        beats = (kc > kr) | ((kc == kr) & (ic < ir))
        rank = rank + jnp.sum(beats.astype(jnp.int32), axis=0, keepdims=True)

    iota_r = lax.broadcasted_iota(jnp.int32, (K, P), 0)
    oh = iota_r == rank
    irf = ir.astype(jnp.float32)
    sel = jnp.sum(jnp.where(oh, jnp.broadcast_to(irf, (K, P)), 0.0), axis=1)

    sel_i = sel.astype(jnp.int32)
    sub_pos = sel_i // N
    obj_pos = sel_i - sub_pos * N

    oh_s = (sub_pos[:, None]
            == lax.broadcasted_iota(jnp.int32, (K, N), 1)).astype(jnp.float32)
    oh_o = (obj_pos[:, None]
            == lax.broadcasted_iota(jnp.int32, (K, N), 1)).astype(jnp.float32)
    dn = (((1,), (0,)), ((), ()))
    sel_sub = lax.dot_general(oh_s, subf_ref[...], dn)
    sel_obj = lax.dot_general(oh_o, objf_ref[...], dn)

    dt = (((1,), (1,)), ((), ()))
    h = jnp.maximum(lax.dot_general(sel_sub, w1a[...], dt)
                    + lax.dot_general(sel_obj, w1b[...], dt) + b1[...], 0.0)
    out_ref[...] = lax.dot_general(h, w2[...], dt) + b2[...]


def _select_call(krow, kcol, irow, icol, subf, objf, w1a, w1b, b1, w2, b2):
    full = lambda s: pl.BlockSpec(s, lambda b: (0,) * len(s))
    return pl.pallas_call(
        _select_body,
        grid=(B,),
        in_specs=[
            pl.BlockSpec((None, 1, P), lambda b: (b, 0, 0)),
            pl.BlockSpec((None, P, 1), lambda b: (b, 0, 0)),
            pl.BlockSpec((None, 1, P), lambda b: (b, 0, 0)),
            pl.BlockSpec((None, P, 1), lambda b: (b, 0, 0)),
            pl.BlockSpec((None, N, D), lambda b: (b, 0, 0)),
            pl.BlockSpec((None, N, D), lambda b: (b, 0, 0)),
            full((D, D)), full((D, D)), full((1, D)),
            full((D, D)), full((1, D)),
        ],
        out_specs=pl.BlockSpec((None, K, D), lambda b: (b, 0, 0)),
        out_shape=jax.ShapeDtypeStruct((B, K, D), jnp.float32),
    )(krow, kcol, irow, icol, subf, objf, w1a, w1b, b1, w2, b2)


def kernel(hs_last, w_sub1, b_sub1, w_sub2, b_sub2, w_obj1, b_obj1,
           w_obj2, b_obj2, conv1_w, conv1_b, conv2_w, conv2_b,
           w_pair1, b_pair1, w_pair2, b_pair2):
    hs_pair = hs_last.reshape(B, N, 2, D)
    subf = hs_pair[:, :, 0, :]
    objf = hs_pair[:, :, 1, :]

    imp = _mlp_call(
        subf, objf,
        w_sub1, b_sub1.reshape(1, D), w_sub2, b_sub2.reshape(1, D),
        w_obj1, b_obj1.reshape(1, D), w_obj2, b_obj2.reshape(1, D))
    skeys = _conv_call(imp, conv1_w.reshape(16, 9), conv1_b,
                       conv2_w.reshape(16), conv2_b)

    t16 = _thresh_call(skeys)
    tvec = jnp.repeat(t16, 2, axis=0)
    sk32 = skeys.reshape(NSUB, N // 2, N)

    gti, gtk, tie = _sc_call(sk32, tvec)

    ts = t16[:, 0]
    ck = jnp.concatenate(
        [gtk.reshape(B, 2 * GBUF),
         jnp.broadcast_to(ts[:, None], (B, 2 * TBUF))], axis=1)
    ci = jnp.concatenate(
        [gti.reshape(B, 2 * GBUF), tie.reshape(B, 2 * TBUF)], axis=1)

    return _select_call(
        ck[:, None, :], ck[:, :, None], ci[:, None, :], ci[:, :, None],
        subf, objf,
        w_pair1[:, :D], w_pair1[:, D:], b_pair1.reshape(1, D),
        w_pair2, b_pair2.reshape(1, D))
